# baseline (speedup 1.0000x reference)
import functools

import jax
import jax.numpy as jnp
from jax import lax
from jax.experimental import pallas as pl
from jax.experimental.pallas import tpu as pltpu

_BN_EPS = 1e-5


def _conv3x3_bf16(slab, w_ref, *, h, w, cin, cout):
    acc = jnp.zeros((h, w, cout), jnp.float32)
    for kh in range(3):
        for kw in range(3):
            xs = slab[kh:kh + h, kw:kw + w, :]
            acc = acc + lax.dot_general(
                xs, w_ref[kh * 3 + kw], (((2,), (0,)), ((), ())),
                preferred_element_type=jnp.float32)
    return acc


def _bn_fold(psum_ref, psq_ref, gamma_ref, beta_ref, m):
    s1 = jnp.sum(psum_ref[...], axis=(0, 1))
    s2 = jnp.sum(psq_ref[...], axis=(0, 1))
    mean = s1 / m
    var = jnp.maximum(s2 / m - mean * mean, 0.0)
    scale = gamma_ref[0] * lax.rsqrt(var + _BN_EPS)
    shift = beta_ref[0] - mean * scale
    return scale, shift


def _stage1_kernel(x_ref, w1_ref, acc1_ref, psum_ref, psq_ref, slab,
                   *, h, w, cin, cout, nb):
    psum = jnp.zeros((1, 1, cout), jnp.float32)
    psq = jnp.zeros((1, 1, cout), jnp.float32)
    for b in range(nb):
        slab[1:h + 1, 1:w + 1, :] = x_ref[b].astype(jnp.bfloat16)
        slab[0:1, :, :] = jnp.zeros((1, w + 2, cin), jnp.bfloat16)
        slab[h + 1:h + 2, :, :] = jnp.zeros((1, w + 2, cin), jnp.bfloat16)
        slab[:, 0:1, :] = jnp.zeros((h + 2, 1, cin), jnp.bfloat16)
        slab[:, w + 1:w + 2, :] = jnp.zeros((h + 2, 1, cin), jnp.bfloat16)
        acc = _conv3x3_bf16(slab[...], w1_ref, h=h, w=w, cin=cin, cout=cout)
        psum = psum + jnp.sum(acc, axis=(0, 1)).reshape(1, 1, cout)
        psq = psq + jnp.sum(acc * acc, axis=(0, 1)).reshape(1, 1, cout)
        acc1_ref[b, 1:h + 1, 1:w + 1, :] = acc.astype(jnp.bfloat16)
    psum_ref[...] = psum
    psq_ref[...] = psq


def _stage2_kernel(acc1_ref, w2_ref, s1sum_ref, s1sq_ref, g1_ref, be1_ref,
                   acc2_ref, psum_ref, psq_ref, *, h, w, cout, m, nb):
    sc1, sh1 = _bn_fold(s1sum_ref, s1sq_ref, g1_ref, be1_ref, m)
    psum = jnp.zeros((1, 1, cout), jnp.float32)
    psq = jnp.zeros((1, 1, cout), jnp.float32)
    for b in range(nb):
        a = acc1_ref[b].astype(jnp.float32)
        act = jnp.maximum(a * sc1 + sh1, 0.0)
        rows = lax.broadcasted_iota(jnp.int32, act.shape, 0)
        cols = lax.broadcasted_iota(jnp.int32, act.shape, 1)
        interior = (rows >= 1) & (rows <= h) & (cols >= 1) & (cols <= w)
        act = jnp.where(interior, act, 0.0).astype(jnp.bfloat16)
        acc = _conv3x3_bf16(act, w2_ref, h=h, w=w, cin=cout, cout=cout)
        psum = psum + jnp.sum(acc, axis=(0, 1)).reshape(1, 1, cout)
        psq = psq + jnp.sum(acc * acc, axis=(0, 1)).reshape(1, 1, cout)
        acc2_ref[b, 1:h + 1, 1:w + 1, :] = acc.astype(jnp.bfloat16)
    psum_ref[...] = psum
    psq_ref[...] = psq


def _stage3_kernel(acc2_ref, x_ref, wres_ref, s2sum_ref, s2sq_ref, g2_ref,
                   be2_ref, bres_ref, out_ref, *, h, w, cin, cout, m, nb):
    sc2, sh2 = _bn_fold(s2sum_ref, s2sq_ref, g2_ref, be2_ref, m)
    for b in range(nb):
        a2 = acc2_ref[b, 1:h + 1, 1:w + 1, :].astype(jnp.float32)
        y = jnp.maximum(a2 * sc2 + sh2, 0.0)
        xs = x_ref[b].reshape(h * w, cin).astype(jnp.bfloat16)
        res = jnp.dot(xs, wres_ref[...],
                      preferred_element_type=jnp.float32) + bres_ref[0]
        out_ref[b] = jnp.maximum(y + res.reshape(h, w, cout), 0.0)


def kernel(x, w1, b1, g1, be1, w2, b2, g2, be2, wres, bres):
    N, Cin, H, W = x.shape
    Cout = w1.shape[-1]
    M = N * H * W
    Hp, Wp = H + 2, W + 2

    NB = 2
    G = N // NB

    xt = jnp.transpose(x, (0, 2, 3, 1))
    w1b = w1.reshape(9, Cin, Cout).astype(jnp.bfloat16)
    w2b = w2.reshape(9, Cout, Cout).astype(jnp.bfloat16)
    wresb = wres.reshape(Cin, Cout).astype(jnp.bfloat16)

    cparams = pltpu.CompilerParams(dimension_semantics=("parallel",),
                                   vmem_limit_bytes=64 * 1024 * 1024)

    def const_spec(shape):
        return pl.BlockSpec(shape, lambda n: (0,) * len(shape))

    img = lambda c, dt: jax.ShapeDtypeStruct((N, Hp, Wp, c), dt)
    img_spec = lambda c: pl.BlockSpec((NB, Hp, Wp, c), lambda n: (n, 0, 0, 0))
    x_spec = pl.BlockSpec((NB, H, W, Cin), lambda n: (n, 0, 0, 0))
    stat_spec = pl.BlockSpec((1, 1, Cout), lambda n: (n, 0, 0))
    stat_shape = jax.ShapeDtypeStruct((G, 1, Cout), jnp.float32)

    acc1, s1sum, s1sq = pl.pallas_call(
        functools.partial(_stage1_kernel, h=H, w=W, cin=Cin, cout=Cout, nb=NB),
        out_shape=(img(Cout, jnp.bfloat16), stat_shape, stat_shape),
        grid=(G,),
        in_specs=[x_spec, const_spec((9, Cin, Cout))],
        out_specs=(img_spec(Cout), stat_spec, stat_spec),
        scratch_shapes=[pltpu.VMEM((Hp, Wp, Cin), jnp.bfloat16)],
        compiler_params=cparams,
    )(xt, w1b)

    acc2, s2sum, s2sq = pl.pallas_call(
        functools.partial(_stage2_kernel, h=H, w=W, cout=Cout, m=float(M),
                          nb=NB),
        out_shape=(img(Cout, jnp.bfloat16), stat_shape, stat_shape),
        grid=(G,),
        in_specs=[img_spec(Cout), const_spec((9, Cout, Cout)),
                  const_spec((G, 1, Cout)), const_spec((G, 1, Cout)),
                  const_spec((1, Cout)), const_spec((1, Cout))],
        out_specs=(img_spec(Cout), stat_spec, stat_spec),
        compiler_params=cparams,
    )(acc1, w2b, s1sum, s1sq, g1, be1)

    out = pl.pallas_call(
        functools.partial(_stage3_kernel, h=H, w=W, cin=Cin, cout=Cout,
                          m=float(M), nb=NB),
        out_shape=jax.ShapeDtypeStruct((N, H, W, Cout), jnp.float32),
        grid=(G,),
        in_specs=[img_spec(Cout), x_spec, const_spec((Cin, Cout)),
                  const_spec((G, 1, Cout)), const_spec((G, 1, Cout)),
                  const_spec((1, Cout)), const_spec((1, Cout)),
                  const_spec((1, Cout))],
        out_specs=pl.BlockSpec((NB, H, W, Cout), lambda n: (n, 0, 0, 0)),
        compiler_params=cparams,
    )(acc2, xt, wresb, s2sum, s2sq, g2, be2, bres)

    return jnp.transpose(out, (0, 3, 1, 2))


# speedup vs baseline: 1.0080x; 1.0080x over previous
import functools

import jax
import jax.numpy as jnp
from jax import lax
from jax.experimental import pallas as pl
from jax.experimental.pallas import tpu as pltpu

_BN_EPS = 1e-5


def _conv3x3_bf16(slab, w_ref, *, h, w, cin, cout):
    acc = jnp.zeros((h, w, cout), jnp.float32)
    for kh in range(3):
        for kw in range(3):
            xs = slab[kh:kh + h, kw:kw + w, :]
            acc = acc + lax.dot_general(
                xs, w_ref[kh * 3 + kw], (((2,), (0,)), ((), ())),
                preferred_element_type=jnp.float32)
    return acc


def _bn_fold(psum_ref, psq_ref, gamma_ref, beta_ref, m):
    s1 = jnp.sum(psum_ref[...], axis=(0, 1))
    s2 = jnp.sum(psq_ref[...], axis=(0, 1))
    mean = s1 / m
    var = jnp.maximum(s2 / m - mean * mean, 0.0)
    scale = gamma_ref[0] * lax.rsqrt(var + _BN_EPS)
    shift = beta_ref[0] - mean * scale
    return scale, shift


def _stage1_kernel(x_ref, w1_ref, acc1_ref, psum_ref, psq_ref, slab,
                   *, h, w, cin, cout):
    slab[1:h + 1, 1:w + 1, :] = x_ref[0].astype(jnp.bfloat16)
    slab[0:1, :, :] = jnp.zeros((1, w + 2, cin), jnp.bfloat16)
    slab[h + 1:h + 2, :, :] = jnp.zeros((1, w + 2, cin), jnp.bfloat16)
    slab[:, 0:1, :] = jnp.zeros((h + 2, 1, cin), jnp.bfloat16)
    slab[:, w + 1:w + 2, :] = jnp.zeros((h + 2, 1, cin), jnp.bfloat16)
    acc = _conv3x3_bf16(slab[...], w1_ref, h=h, w=w, cin=cin, cout=cout)
    psum_ref[...] = jnp.sum(acc, axis=(0, 1)).reshape(1, 1, cout)
    psq_ref[...] = jnp.sum(acc * acc, axis=(0, 1)).reshape(1, 1, cout)
    acc1_ref[0] = acc.astype(jnp.bfloat16)


def _stage2_kernel(acc1_ref, w2_ref, s1sum_ref, s1sq_ref, g1_ref, be1_ref,
                   acc2_ref, psum_ref, psq_ref, slab, *, h, w, cout, m):
    sc1, sh1 = _bn_fold(s1sum_ref, s1sq_ref, g1_ref, be1_ref, m)
    a = acc1_ref[0].astype(jnp.float32)
    slab[1:h + 1, 1:w + 1, :] = jnp.maximum(a * sc1 + sh1, 0.0).astype(jnp.bfloat16)
    slab[0:1, :, :] = jnp.zeros((1, w + 2, cout), jnp.bfloat16)
    slab[h + 1:h + 2, :, :] = jnp.zeros((1, w + 2, cout), jnp.bfloat16)
    slab[:, 0:1, :] = jnp.zeros((h + 2, 1, cout), jnp.bfloat16)
    slab[:, w + 1:w + 2, :] = jnp.zeros((h + 2, 1, cout), jnp.bfloat16)
    acc = _conv3x3_bf16(slab[...], w2_ref, h=h, w=w, cin=cout, cout=cout)
    psum_ref[...] = jnp.sum(acc, axis=(0, 1)).reshape(1, 1, cout)
    psq_ref[...] = jnp.sum(acc * acc, axis=(0, 1)).reshape(1, 1, cout)
    acc2_ref[0] = acc.astype(jnp.bfloat16)


def _stage3_kernel(acc2_ref, x_ref, wres_ref, s2sum_ref, s2sq_ref, g2_ref,
                   be2_ref, bres_ref, out_ref, *, h, w, cin, cout, m):
    sc2, sh2 = _bn_fold(s2sum_ref, s2sq_ref, g2_ref, be2_ref, m)
    a2 = acc2_ref[0].astype(jnp.float32)
    y = jnp.maximum(a2 * sc2 + sh2, 0.0)
    xs = x_ref[0].reshape(h * w, cin).astype(jnp.bfloat16)
    res = jnp.dot(xs, wres_ref[...],
                  preferred_element_type=jnp.float32) + bres_ref[0]
    out_ref[0] = jnp.maximum(y + res.reshape(h, w, cout), 0.0)


def kernel(x, w1, b1, g1, be1, w2, b2, g2, be2, wres, bres):
    N, Cin, H, W = x.shape
    Cout = w1.shape[-1]
    M = N * H * W
    Hp, Wp = H + 2, W + 2

    xt = jnp.transpose(x, (0, 2, 3, 1))
    w1b = w1.reshape(9, Cin, Cout).astype(jnp.bfloat16)
    w2b = w2.reshape(9, Cout, Cout).astype(jnp.bfloat16)
    wresb = wres.reshape(Cin, Cout).astype(jnp.bfloat16)

    cparams = pltpu.CompilerParams(dimension_semantics=("parallel",),
                                   vmem_limit_bytes=64 * 1024 * 1024)

    def const_spec(shape):
        return pl.BlockSpec(shape, lambda n: (0,) * len(shape))

    img = lambda c, dt: jax.ShapeDtypeStruct((N, H, W, c), dt)
    img_spec = lambda c: pl.BlockSpec((1, H, W, c), lambda n: (n, 0, 0, 0))
    x_spec = pl.BlockSpec((1, H, W, Cin), lambda n: (n, 0, 0, 0))
    stat_spec = pl.BlockSpec((1, 1, Cout), lambda n: (n, 0, 0))
    stat_shape = jax.ShapeDtypeStruct((N, 1, Cout), jnp.float32)

    acc1, s1sum, s1sq = pl.pallas_call(
        functools.partial(_stage1_kernel, h=H, w=W, cin=Cin, cout=Cout),
        out_shape=(img(Cout, jnp.bfloat16), stat_shape, stat_shape),
        grid=(N,),
        in_specs=[x_spec, const_spec((9, Cin, Cout))],
        out_specs=(img_spec(Cout), stat_spec, stat_spec),
        scratch_shapes=[pltpu.VMEM((Hp, Wp, Cin), jnp.bfloat16)],
        compiler_params=cparams,
    )(xt, w1b)

    acc2, s2sum, s2sq = pl.pallas_call(
        functools.partial(_stage2_kernel, h=H, w=W, cout=Cout, m=float(M)),
        out_shape=(img(Cout, jnp.bfloat16), stat_shape, stat_shape),
        grid=(N,),
        in_specs=[img_spec(Cout), const_spec((9, Cout, Cout)),
                  const_spec((N, 1, Cout)), const_spec((N, 1, Cout)),
                  const_spec((1, Cout)), const_spec((1, Cout))],
        out_specs=(img_spec(Cout), stat_spec, stat_spec),
        scratch_shapes=[pltpu.VMEM((Hp, Wp, Cout), jnp.bfloat16)],
        compiler_params=cparams,
    )(acc1, w2b, s1sum, s1sq, g1, be1)

    out = pl.pallas_call(
        functools.partial(_stage3_kernel, h=H, w=W, cin=Cin, cout=Cout,
                          m=float(M)),
        out_shape=jax.ShapeDtypeStruct((N, H, W, Cout), jnp.float32),
        grid=(N,),
        in_specs=[img_spec(Cout), x_spec, const_spec((Cin, Cout)),
                  const_spec((N, 1, Cout)), const_spec((N, 1, Cout)),
                  const_spec((1, Cout)), const_spec((1, Cout)),
                  const_spec((1, Cout))],
        out_specs=pl.BlockSpec((1, H, W, Cout), lambda n: (n, 0, 0, 0)),
        compiler_params=cparams,
    )(acc2, xt, wresb, s2sum, s2sq, g2, be2, bres)

    return jnp.transpose(out, (0, 3, 1, 2))


# speedup vs baseline: 1.0480x; 1.0397x over previous
import functools

import jax
import jax.numpy as jnp
from jax import lax
from jax.experimental import pallas as pl
from jax.experimental.pallas import tpu as pltpu

_BN_EPS = 1e-5


def _conv3x3_bf16(slab, w_ref, *, h, w, cin, cout):
    acc = jnp.zeros((h, w, cout), jnp.float32)
    for kh in range(3):
        for kw in range(3):
            xs = slab[kh:kh + h, kw:kw + w, :]
            acc = acc + lax.dot_general(
                xs, w_ref[kh * 3 + kw], (((2,), (0,)), ((), ())),
                preferred_element_type=jnp.float32)
    return acc


def _bn_fold(psum_ref, psq_ref, gamma_ref, beta_ref, m):
    s1 = jnp.sum(psum_ref[...], axis=(0, 1))
    s2 = jnp.sum(psq_ref[...], axis=(0, 1))
    mean = s1 / m
    var = jnp.maximum(s2 / m - mean * mean, 0.0)
    scale = gamma_ref[0] * lax.rsqrt(var + _BN_EPS)
    shift = beta_ref[0] - mean * scale
    return scale, shift


def _stage1_kernel(x_ref, w1_ref, acc1_ref, psum_ref, psq_ref, slab,
                   *, h, w, cin, cout):
    slab[1:h + 1, 1:w + 1, :] = x_ref[0].astype(jnp.bfloat16)
    slab[0:1, :, :] = jnp.zeros((1, w + 2, cin), jnp.bfloat16)
    slab[h + 1:h + 2, :, :] = jnp.zeros((1, w + 2, cin), jnp.bfloat16)
    slab[:, 0:1, :] = jnp.zeros((h + 2, 1, cin), jnp.bfloat16)
    slab[:, w + 1:w + 2, :] = jnp.zeros((h + 2, 1, cin), jnp.bfloat16)
    acc = _conv3x3_bf16(slab[...], w1_ref, h=h, w=w, cin=cin, cout=cout)
    psum_ref[...] = jnp.sum(acc, axis=(0, 1)).reshape(1, 1, cout)
    psq_ref[...] = jnp.sum(acc * acc, axis=(0, 1)).reshape(1, 1, cout)
    acc1_ref[0, 1:h + 1, 1:w + 1, :] = acc.astype(jnp.bfloat16)


def _stage2_kernel(acc1_ref, w2_ref, s1sum_ref, s1sq_ref, g1_ref, be1_ref,
                   acc2_ref, psum_ref, psq_ref, *, h, w, cout, m):
    sc1, sh1 = _bn_fold(s1sum_ref, s1sq_ref, g1_ref, be1_ref, m)
    a = acc1_ref[0].astype(jnp.float32)
    act = jnp.maximum(a * sc1 + sh1, 0.0)
    rows = lax.broadcasted_iota(jnp.int32, act.shape, 0)
    cols = lax.broadcasted_iota(jnp.int32, act.shape, 1)
    interior = (rows >= 1) & (rows <= h) & (cols >= 1) & (cols <= w)
    act = jnp.where(interior, act, 0.0).astype(jnp.bfloat16)
    acc = _conv3x3_bf16(act, w2_ref, h=h, w=w, cin=cout, cout=cout)
    psum_ref[...] = jnp.sum(acc, axis=(0, 1)).reshape(1, 1, cout)
    psq_ref[...] = jnp.sum(acc * acc, axis=(0, 1)).reshape(1, 1, cout)
    acc2_ref[0, 1:h + 1, 1:w + 1, :] = acc.astype(jnp.bfloat16)


def _stage3_kernel(acc2_ref, x_ref, wres_ref, s2sum_ref, s2sq_ref, g2_ref,
                   be2_ref, bres_ref, out_ref, *, h, w, cin, cout, m):
    sc2, sh2 = _bn_fold(s2sum_ref, s2sq_ref, g2_ref, be2_ref, m)
    a2 = acc2_ref[0, 1:h + 1, 1:w + 1, :].astype(jnp.float32)
    y = jnp.maximum(a2 * sc2 + sh2, 0.0)
    xs = x_ref[0].reshape(h * w, cin).astype(jnp.bfloat16)
    res = jnp.dot(xs, wres_ref[...],
                  preferred_element_type=jnp.float32) + bres_ref[0]
    out_ref[0] = jnp.maximum(y + res.reshape(h, w, cout), 0.0)


def kernel(x, w1, b1, g1, be1, w2, b2, g2, be2, wres, bres):
    N, Cin, H, W = x.shape
    Cout = w1.shape[-1]
    M = N * H * W
    Hp, Wp = H + 2, W + 2

    xt = jnp.transpose(x, (0, 2, 3, 1))
    w1b = w1.reshape(9, Cin, Cout).astype(jnp.bfloat16)
    w2b = w2.reshape(9, Cout, Cout).astype(jnp.bfloat16)
    wresb = wres.reshape(Cin, Cout).astype(jnp.bfloat16)

    cparams = pltpu.CompilerParams(dimension_semantics=("parallel",),
                                   vmem_limit_bytes=64 * 1024 * 1024)

    def const_spec(shape):
        return pl.BlockSpec(shape, lambda n: (0,) * len(shape))

    img = lambda c, dt: jax.ShapeDtypeStruct((N, Hp, Wp, c), dt)
    img_spec = lambda c: pl.BlockSpec((1, Hp, Wp, c), lambda n: (n, 0, 0, 0))
    x_spec = pl.BlockSpec((1, H, W, Cin), lambda n: (n, 0, 0, 0))
    stat_spec = pl.BlockSpec((1, 1, Cout), lambda n: (n, 0, 0))
    stat_shape = jax.ShapeDtypeStruct((N, 1, Cout), jnp.float32)

    acc1, s1sum, s1sq = pl.pallas_call(
        functools.partial(_stage1_kernel, h=H, w=W, cin=Cin, cout=Cout),
        out_shape=(img(Cout, jnp.bfloat16), stat_shape, stat_shape),
        grid=(N,),
        in_specs=[x_spec, const_spec((9, Cin, Cout))],
        out_specs=(img_spec(Cout), stat_spec, stat_spec),
        scratch_shapes=[pltpu.VMEM((Hp, Wp, Cin), jnp.bfloat16)],
        compiler_params=cparams,
    )(xt, w1b)

    acc2, s2sum, s2sq = pl.pallas_call(
        functools.partial(_stage2_kernel, h=H, w=W, cout=Cout, m=float(M)),
        out_shape=(img(Cout, jnp.bfloat16), stat_shape, stat_shape),
        grid=(N,),
        in_specs=[img_spec(Cout), const_spec((9, Cout, Cout)),
                  const_spec((N, 1, Cout)), const_spec((N, 1, Cout)),
                  const_spec((1, Cout)), const_spec((1, Cout))],
        out_specs=(img_spec(Cout), stat_spec, stat_spec),
        compiler_params=cparams,
    )(acc1, w2b, s1sum, s1sq, g1, be1)

    out = pl.pallas_call(
        functools.partial(_stage3_kernel, h=H, w=W, cin=Cin, cout=Cout,
                          m=float(M)),
        out_shape=jax.ShapeDtypeStruct((N, H, W, Cout), jnp.float32),
        grid=(N,),
        in_specs=[img_spec(Cout), x_spec, const_spec((Cin, Cout)),
                  const_spec((N, 1, Cout)), const_spec((N, 1, Cout)),
                  const_spec((1, Cout)), const_spec((1, Cout)),
                  const_spec((1, Cout))],
        out_specs=pl.BlockSpec((1, H, W, Cout), lambda n: (n, 0, 0, 0)),
        compiler_params=cparams,
    )(acc2, xt, wresb, s2sum, s2sq, g2, be2, bres)

    return jnp.transpose(out, (0, 3, 1, 2))


# speedup vs baseline: 1.0991x; 1.0488x over previous
import functools

import jax
import jax.numpy as jnp
from jax import lax
from jax.experimental import pallas as pl
from jax.experimental.pallas import tpu as pltpu

_BN_EPS = 1e-5


def _conv3x3_bf16(slab, w_ref, *, h, w, cin, cout):
    acc = jnp.zeros((h, w, cout), jnp.float32)
    for kh in range(3):
        for kw in range(3):
            xs = slab[kh:kh + h, kw:kw + w, :]
            acc = acc + lax.dot_general(
                xs, w_ref[kh * 3 + kw], (((2,), (0,)), ((), ())),
                preferred_element_type=jnp.float32)
    return acc


def _bn_fold(psum_ref, psq_ref, gamma_ref, beta_ref, m):
    s1 = jnp.sum(psum_ref[...], axis=(0, 1))
    s2 = jnp.sum(psq_ref[...], axis=(0, 1))
    mean = s1 / m
    var = jnp.maximum(s2 / m - mean * mean, 0.0)
    scale = gamma_ref[0] * lax.rsqrt(var + _BN_EPS)
    shift = beta_ref[0] - mean * scale
    return scale, shift


def _stage1_kernel(x_ref, w1_ref, acc1_ref, psum_ref, psq_ref, slab,
                   *, h, w, cin, cout):
    slab[1:h + 1, 1:w + 1, :] = x_ref[0].astype(jnp.bfloat16)
    slab[0:1, :, :] = jnp.zeros((1, w + 2, cin), jnp.bfloat16)
    slab[h + 1:h + 2, :, :] = jnp.zeros((1, w + 2, cin), jnp.bfloat16)
    slab[:, 0:1, :] = jnp.zeros((h + 2, 1, cin), jnp.bfloat16)
    slab[:, w + 1:w + 2, :] = jnp.zeros((h + 2, 1, cin), jnp.bfloat16)
    acc = _conv3x3_bf16(slab[...], w1_ref, h=h, w=w, cin=cin, cout=cout)
    psum_ref[...] = jnp.sum(acc, axis=(0, 1)).reshape(1, 1, cout)
    psq_ref[...] = jnp.sum(acc * acc, axis=(0, 1)).reshape(1, 1, cout)
    acc1_ref[0, 1:h + 1, 1:w + 1, :] = acc.astype(jnp.bfloat16)


def _stage2_kernel(acc1_ref, w2_ref, s1sum_ref, s1sq_ref, g1_ref, be1_ref,
                   acc2_ref, psum_ref, psq_ref, *, h, w, cout, m):
    sc1, sh1 = _bn_fold(s1sum_ref, s1sq_ref, g1_ref, be1_ref, m)
    a = acc1_ref[0].astype(jnp.float32)
    act = jnp.maximum(a * sc1 + sh1, 0.0)
    rows = lax.broadcasted_iota(jnp.int32, act.shape, 0)
    cols = lax.broadcasted_iota(jnp.int32, act.shape, 1)
    interior = (rows >= 1) & (rows <= h) & (cols >= 1) & (cols <= w)
    act = jnp.where(interior, act, 0.0).astype(jnp.bfloat16)
    acc = _conv3x3_bf16(act, w2_ref, h=h, w=w, cin=cout, cout=cout)
    psum_ref[...] = jnp.sum(acc, axis=(0, 1)).reshape(1, 1, cout)
    psq_ref[...] = jnp.sum(acc * acc, axis=(0, 1)).reshape(1, 1, cout)
    acc2_ref[0] = acc.astype(jnp.bfloat16)


def _stage3_kernel(acc2_ref, x_ref, wres_ref, s2sum_ref, s2sq_ref, g2_ref,
                   be2_ref, bres_ref, out_ref, *, h, w, cin, cout, m):
    sc2, sh2 = _bn_fold(s2sum_ref, s2sq_ref, g2_ref, be2_ref, m)
    a2 = acc2_ref[0].astype(jnp.float32)
    y = jnp.maximum(a2 * sc2 + sh2, 0.0)
    xs = x_ref[0].reshape(h * w, cin).astype(jnp.bfloat16)
    res = jnp.dot(xs, wres_ref[...],
                  preferred_element_type=jnp.float32) + bres_ref[0]
    out_ref[0] = jnp.maximum(y + res.reshape(h, w, cout), 0.0)


def kernel(x, w1, b1, g1, be1, w2, b2, g2, be2, wres, bres):
    N, Cin, H, W = x.shape
    Cout = w1.shape[-1]
    M = N * H * W
    Hp, Wp = H + 2, W + 2

    xt = jnp.transpose(x, (0, 2, 3, 1))
    w1b = w1.reshape(9, Cin, Cout).astype(jnp.bfloat16)
    w2b = w2.reshape(9, Cout, Cout).astype(jnp.bfloat16)
    wresb = wres.reshape(Cin, Cout).astype(jnp.bfloat16)

    cparams = pltpu.CompilerParams(dimension_semantics=("parallel",),
                                   vmem_limit_bytes=64 * 1024 * 1024)

    def const_spec(shape):
        return pl.BlockSpec(shape, lambda n: (0,) * len(shape))

    img = lambda c, dt: jax.ShapeDtypeStruct((N, Hp, Wp, c), dt)
    img_spec = lambda c: pl.BlockSpec((1, Hp, Wp, c), lambda n: (n, 0, 0, 0))
    x_spec = pl.BlockSpec((1, H, W, Cin), lambda n: (n, 0, 0, 0))
    stat_spec = pl.BlockSpec((1, 1, Cout), lambda n: (n, 0, 0))
    stat_shape = jax.ShapeDtypeStruct((N, 1, Cout), jnp.float32)

    acc1, s1sum, s1sq = pl.pallas_call(
        functools.partial(_stage1_kernel, h=H, w=W, cin=Cin, cout=Cout),
        out_shape=(img(Cout, jnp.bfloat16), stat_shape, stat_shape),
        grid=(N,),
        in_specs=[x_spec, const_spec((9, Cin, Cout))],
        out_specs=(img_spec(Cout), stat_spec, stat_spec),
        scratch_shapes=[pltpu.VMEM((Hp, Wp, Cin), jnp.bfloat16)],
        compiler_params=cparams,
    )(xt, w1b)

    dense_spec = pl.BlockSpec((1, H, W, Cout), lambda n: (n, 0, 0, 0))
    acc2, s2sum, s2sq = pl.pallas_call(
        functools.partial(_stage2_kernel, h=H, w=W, cout=Cout, m=float(M)),
        out_shape=(jax.ShapeDtypeStruct((N, H, W, Cout), jnp.bfloat16),
                   stat_shape, stat_shape),
        grid=(N,),
        in_specs=[img_spec(Cout), const_spec((9, Cout, Cout)),
                  const_spec((N, 1, Cout)), const_spec((N, 1, Cout)),
                  const_spec((1, Cout)), const_spec((1, Cout))],
        out_specs=(dense_spec, stat_spec, stat_spec),
        compiler_params=cparams,
    )(acc1, w2b, s1sum, s1sq, g1, be1)

    out = pl.pallas_call(
        functools.partial(_stage3_kernel, h=H, w=W, cin=Cin, cout=Cout,
                          m=float(M)),
        out_shape=jax.ShapeDtypeStruct((N, H, W, Cout), jnp.float32),
        grid=(N,),
        in_specs=[dense_spec, x_spec, const_spec((Cin, Cout)),
                  const_spec((N, 1, Cout)), const_spec((N, 1, Cout)),
                  const_spec((1, Cout)), const_spec((1, Cout)),
                  const_spec((1, Cout))],
        out_specs=pl.BlockSpec((1, H, W, Cout), lambda n: (n, 0, 0, 0)),
        compiler_params=cparams,
    )(acc2, xt, wresb, s2sum, s2sq, g2, be2, bres)

    return jnp.transpose(out, (0, 3, 1, 2))
